# Initial kernel scaffold; baseline (speedup 1.0000x reference)
import functools

import jax
import jax.numpy as jnp
from jax import lax
from jax.experimental import pallas as pl
from jax.experimental.pallas import tpu as pltpu
from jax.experimental.pallas import tpu_sc as plsc

_N = 10000
_E = 320000
_D = 128
_L = 3
_AF = 9
_BF = 3
_AV = 100
_BV = 8

_C = 128
_NCHUNKS = _E // _C
_NW = 32
_BASE_CHUNKS = _NCHUNKS // _NW
_EXTRA = _NCHUNKS - _BASE_CHUNKS * _NW
_RPT = _N // 16


def _encode_body(x_ref, ea_ref, aemb_ref, bemb_ref, h_ref, combo_ref, ctab_ref):
    h = jnp.zeros((_N, _D), jnp.float32)
    for i in range(_AF):
        xi = x_ref[:, i:i + 1]
        oh = (xi == lax.broadcasted_iota(jnp.int32, (1, _AV), 1)).astype(jnp.float32)
        h = h + jnp.dot(oh, aemb_ref[i], preferred_element_type=jnp.float32)
    h_ref[...] = h

    combo_ref[...] = ea_ref[0] * 64 + ea_ref[1] * 8 + ea_ref[2]

    cidx = lax.broadcasted_iota(jnp.int32, (512, 1), 0)
    for l in range(_L):
        acc = jnp.zeros((512, _D), jnp.float32)
        for j in range(_BF):
            shift = 3 * (_BF - 1 - j)
            digit = (cidx >> shift) & 7
            oh = (digit == lax.broadcasted_iota(jnp.int32, (1, _BV), 1)).astype(jnp.float32)
            acc = acc + jnp.dot(oh, bemb_ref[l, j], preferred_element_type=jnp.float32)
        ctab_ref[l] = acc


_encode_call = pl.pallas_call(
    _encode_body,
    out_shape=(
        jax.ShapeDtypeStruct((_N, _D), jnp.float32),
        jax.ShapeDtypeStruct((_NCHUNKS, _C), jnp.int32),
        jax.ShapeDtypeStruct((_L, 512, _D), jnp.float32),
    ),
)


def _edge_body(h_hbm, src_hbm, dst_hbm, combo_hbm, ctab_hbm, zeros_hbm, out_hbm,
               srcidx_v, dstidx_v, comboidx_v, rows_v, ctab_v, agg_sh, gsem):
    c = lax.axis_index("c")
    s = lax.axis_index("s")
    wid = s * 2 + c
    tid = s

    pltpu.sync_copy(ctab_hbm, ctab_v)
    pltpu.sync_copy(zeros_hbm.at[pl.ds(tid * _RPT, _RPT)],
                    agg_sh.at[pl.ds(tid * _RPT, _RPT)])
    plsc.subcore_barrier()

    nchunks = _BASE_CHUNKS + jnp.where(wid < _EXTRA, 1, 0)

    def chunk_body(k, carry):
        off = (wid + _NW * k) * _C
        pltpu.sync_copy(src_hbm.at[pl.ds(off, _C)], srcidx_v)
        pltpu.sync_copy(dst_hbm.at[pl.ds(off, _C)], dstidx_v)
        pltpu.sync_copy(combo_hbm.at[pl.ds(off, _C)], comboidx_v)
        pltpu.async_copy(h_hbm.at[srcidx_v], rows_v, gsem).wait()

        def row_body(i, carry2):
            ci = comboidx_v[i]
            for g in range(8):
                sl = pl.ds(g * 16, 16)
                rows_v[i, sl] = jnp.maximum(rows_v[i, sl] + ctab_v[ci, sl], 0.0)
            return carry2

        lax.fori_loop(0, _C, row_body, 0)
        pltpu.sync_copy(rows_v, agg_sh.at[dstidx_v], add=True)
        return carry

    lax.fori_loop(0, nchunks, chunk_body, 0)
    plsc.subcore_barrier()

    pltpu.sync_copy(agg_sh.at[pl.ds(tid * _RPT, _RPT)],
                    out_hbm.at[c, pl.ds(tid * _RPT, _RPT)])


_edge_call = pl.kernel(
    _edge_body,
    out_type=jax.ShapeDtypeStruct((2, _N, _D), jnp.float32),
    mesh=plsc.VectorSubcoreMesh(core_axis_name="c", subcore_axis_name="s"),
    scratch_types=[
        pltpu.VMEM((_C,), jnp.int32),
        pltpu.VMEM((_C,), jnp.int32),
        pltpu.VMEM((_C,), jnp.int32),
        pltpu.VMEM((_C, _D), jnp.float32),
        pltpu.VMEM((512, _D), jnp.float32),
        pltpu.VMEM_SHARED((_N, _D), jnp.float32),
        pltpu.SemaphoreType.DMA,
    ],
)


def _mlp_body(relu_out, scale_ref, h_ref, a0_ref, a1_ref, w1_ref, b1_ref,
              g1_ref, be1_ref, w2_ref, b2_ref, bng_ref, bnbe_ref, out_ref):
    z = scale_ref[...] * h_ref[...] + a0_ref[...] + a1_ref[...]
    z = jnp.dot(z, w1_ref[...], preferred_element_type=jnp.float32) + b1_ref[...]
    mu = jnp.mean(z, axis=0, keepdims=True)
    zc = z - mu
    var = jnp.mean(zc * zc, axis=0, keepdims=True)
    z = g1_ref[...] * zc * lax.rsqrt(var + 1e-5) + be1_ref[...]
    z = jnp.maximum(z, 0.0)
    z = jnp.dot(z, w2_ref[...], preferred_element_type=jnp.float32) + b2_ref[...]
    mu2 = jnp.mean(z, axis=0, keepdims=True)
    zc2 = z - mu2
    var2 = jnp.mean(zc2 * zc2, axis=0, keepdims=True)
    z = bng_ref[...] * zc2 * lax.rsqrt(var2 + 1e-5) + bnbe_ref[...]
    if relu_out:
        z = jnp.maximum(z, 0.0)
    out_ref[...] = z


def _mlp_call(relu_out):
    return pl.pallas_call(
        functools.partial(_mlp_body, relu_out),
        out_shape=jax.ShapeDtypeStruct((_N, _D), jnp.float32),
    )


def kernel(x, edge_index, edge_attr, batch, atom_emb, bond_emb, W1, b1, g1,
           be1, W2, b2, eps_p, bn_g, bn_be):
    del batch
    ea3 = edge_attr.T.reshape(_BF, _NCHUNKS, _C)
    h, combo2d, ctab = _encode_call(x, ea3, atom_emb, bond_emb)
    combo = combo2d.reshape(_E)
    src = edge_index[0]
    dst = edge_index[1]
    zeros = jnp.zeros((_N, _D), jnp.float32)
    for l in range(_L):
        agg2 = _edge_call(h, src, dst, combo, ctab[l], zeros)
        scale = (1.0 + eps_p[l]).reshape(1, 1)
        h = _mlp_call(l < _L - 1)(
            scale, h, agg2[0], agg2[1], W1[l], b1[l].reshape(1, _D),
            g1[l].reshape(1, _D), be1[l].reshape(1, _D), W2[l],
            b2[l].reshape(1, _D), bn_g[l].reshape(1, _D), bn_be[l].reshape(1, _D))
    return h


# speedup vs baseline: 6.2187x; 6.2187x over previous
import functools

import jax
import jax.numpy as jnp
from jax import lax
from jax.experimental import pallas as pl
from jax.experimental.pallas import tpu as pltpu
from jax.experimental.pallas import tpu_sc as plsc

_N = 10000
_E = 320000
_D = 128
_L = 3
_AF = 9
_BF = 3
_AV = 100
_BV = 8

_C = 128
_NCHUNKS = _E // _C
_NW = 32
_BASE_CHUNKS = _NCHUNKS // _NW
_EXTRA = _NCHUNKS - _BASE_CHUNKS * _NW
_NP = 10240
_RPT = _NP // 16


_NB = 1000


def _encode_body(x_ref, ea_ref, aemb_ref, bemb_ref, h_ref, combo_ref, ctab_ref):
    h = jnp.zeros((_NB, _D), jnp.float32)
    for i in range(_AF):
        xi = x_ref[:, i:i + 1]
        oh = (xi == lax.broadcasted_iota(jnp.int32, (1, _AV), 1)).astype(jnp.float32)
        h = h + jnp.dot(oh, aemb_ref[i], preferred_element_type=jnp.float32,
                        precision=lax.Precision.HIGHEST)
    h_ref[...] = h

    @pl.when(pl.program_id(0) == 0)
    def _tables():
        combo_ref[...] = ea_ref[0] * 64 + ea_ref[1] * 8 + ea_ref[2]
        cidx = lax.broadcasted_iota(jnp.int32, (512, 1), 0)
        for l in range(_L):
            acc = jnp.zeros((512, _D), jnp.float32)
            for j in range(_BF):
                shift = 3 * (_BF - 1 - j)
                digit = (cidx >> shift) & 7
                term = jnp.zeros((512, _D), jnp.float32)
                for v in range(_BV):
                    term = jnp.where(digit == v, bemb_ref[l, j, v][None, :], term)
                acc = acc + term
            ctab_ref[l] = acc


_encode_call = pl.pallas_call(
    _encode_body,
    grid=(_N // _NB,),
    in_specs=[
        pl.BlockSpec((_NB, _AF), lambda i: (i, 0)),
        pl.BlockSpec((_BF, _NCHUNKS, _C), lambda i: (0, 0, 0)),
        pl.BlockSpec((_AF, _AV, _D), lambda i: (0, 0, 0)),
        pl.BlockSpec((_L, _BF, _BV, _D), lambda i: (0, 0, 0, 0)),
    ],
    out_specs=(
        pl.BlockSpec((_NB, _D), lambda i: (i, 0)),
        pl.BlockSpec((_NCHUNKS, _C), lambda i: (0, 0)),
        pl.BlockSpec((_L, 512, _D), lambda i: (0, 0, 0)),
    ),
    out_shape=(
        jax.ShapeDtypeStruct((_N, _D), jnp.float32),
        jax.ShapeDtypeStruct((_NCHUNKS, _C), jnp.int32),
        jax.ShapeDtypeStruct((_L, 512, _D), jnp.float32),
    ),
)


def _edge_body(h_hbm, src_hbm, dst_hbm, combo_hbm, ctab_hbm, zeros_hbm, out_hbm,
               srcidx_v, dstidx_v, comboidx_v, rows_v, agg_sh, gsem, csem):
    c = lax.axis_index("c")
    s = lax.axis_index("s")
    wid = s * 2 + c
    tid = s

    pltpu.sync_copy(zeros_hbm.at[pl.ds(tid * _RPT, _RPT)],
                    agg_sh.at[pl.ds(tid * _RPT, _RPT)])
    plsc.subcore_barrier()

    nchunks = _BASE_CHUNKS + jnp.where(wid < _EXTRA, 1, 0)

    def chunk_body(k, carry):
        off = (wid + _NW * k) * _C
        pltpu.sync_copy(src_hbm.at[pl.ds(off, _C)], srcidx_v)
        pltpu.sync_copy(dst_hbm.at[pl.ds(off, _C)], dstidx_v)
        pltpu.sync_copy(combo_hbm.at[pl.ds(off, _C)], comboidx_v)
        pltpu.async_copy(h_hbm.at[srcidx_v], rows_v, gsem).wait()
        pltpu.async_copy(ctab_hbm.at[comboidx_v], rows_v, csem, add=True).wait()

        def row_body(i, carry2):
            for g in range(8):
                sl = pl.ds(g * 16, 16)
                rows_v[i, sl] = jnp.maximum(rows_v[i, sl], 0.0)
            return carry2

        lax.fori_loop(0, _C, row_body, 0)
        pltpu.sync_copy(rows_v, agg_sh.at[dstidx_v], add=True)
        return carry

    lax.fori_loop(0, nchunks, chunk_body, 0)
    plsc.subcore_barrier()

    pltpu.sync_copy(agg_sh.at[pl.ds(tid * _RPT, _RPT)],
                    out_hbm.at[c, pl.ds(tid * _RPT, _RPT)])


@functools.cache
def _edge_call():
    return pl.kernel(
        _edge_body,
        out_type=jax.ShapeDtypeStruct((2, _NP, _D), jnp.float32),
        mesh=plsc.VectorSubcoreMesh(core_axis_name="c", subcore_axis_name="s"),
        scratch_types=[
            pltpu.VMEM((_C,), jnp.int32),
            pltpu.VMEM((_C,), jnp.int32),
            pltpu.VMEM((_C,), jnp.int32),
            pltpu.VMEM((_C, _D), jnp.float32),
            pltpu.VMEM_SHARED((_NP, _D), jnp.float32),
            pltpu.SemaphoreType.DMA,
            pltpu.SemaphoreType.DMA,
        ],
    )


def _mlp_body(relu_out, scale_ref, h_ref, a0_ref, a1_ref, w1_ref, b1_ref,
              g1_ref, be1_ref, w2_ref, b2_ref, bng_ref, bnbe_ref, out_ref):
    z = scale_ref[...] * h_ref[...] + a0_ref[...] + a1_ref[...]
    z = jnp.dot(z, w1_ref[...], preferred_element_type=jnp.float32) + b1_ref[...]
    mu = jnp.mean(z, axis=0, keepdims=True)
    zc = z - mu
    var = jnp.mean(zc * zc, axis=0, keepdims=True)
    z = g1_ref[...] * zc * lax.rsqrt(var + 1e-5) + be1_ref[...]
    z = jnp.maximum(z, 0.0)
    z = jnp.dot(z, w2_ref[...], preferred_element_type=jnp.float32) + b2_ref[...]
    mu2 = jnp.mean(z, axis=0, keepdims=True)
    zc2 = z - mu2
    var2 = jnp.mean(zc2 * zc2, axis=0, keepdims=True)
    z = bng_ref[...] * zc2 * lax.rsqrt(var2 + 1e-5) + bnbe_ref[...]
    if relu_out:
        z = jnp.maximum(z, 0.0)
    out_ref[...] = z


def _mlp_call(relu_out):
    return pl.pallas_call(
        functools.partial(_mlp_body, relu_out),
        out_shape=jax.ShapeDtypeStruct((_N, _D), jnp.float32),
    )


def kernel(x, edge_index, edge_attr, batch, atom_emb, bond_emb, W1, b1, g1,
           be1, W2, b2, eps_p, bn_g, bn_be):
    del batch
    ea3 = edge_attr.T.reshape(_BF, _NCHUNKS, _C)
    h, combo2d, ctab = _encode_call(x, ea3, atom_emb, bond_emb)
    combo = combo2d.reshape(_E)
    src = edge_index[0]
    dst = edge_index[1]
    zeros = jnp.zeros((_NP, _D), jnp.float32)
    for l in range(_L):
        agg2 = _edge_call()(h, src, dst, combo, ctab[l], zeros)
        scale = (1.0 + eps_p[l]).reshape(1, 1)
        h = _mlp_call(l < _L - 1)(
            scale, h, agg2[0, :_N], agg2[1, :_N], W1[l], b1[l].reshape(1, _D),
            g1[l].reshape(1, _D), be1[l].reshape(1, _D), W2[l],
            b2[l].reshape(1, _D), bn_g[l].reshape(1, _D), bn_be[l].reshape(1, _D))
    return h


# speedup vs baseline: 11.1009x; 1.7851x over previous
import functools

import jax
import jax.numpy as jnp
from jax import lax
from jax.experimental import pallas as pl
from jax.experimental.pallas import tpu as pltpu
from jax.experimental.pallas import tpu_sc as plsc

_N = 10000
_E = 320000
_D = 128
_L = 3
_AF = 9
_BF = 3
_AV = 100
_BV = 8

_C = 128
_NCHUNKS = _E // _C
_NW = 32
_CPW = 80
_NCH_P = _NW * _CPW
_EP = _NCH_P * _C
_NBUF = 4
_NP = 10240
_RPT = _NP // 16


_NB = 1000


def _encode_body(x_ref, ea_ref, aemb_ref, bemb_ref, h_ref, combo_ref, ctab_ref):
    h = jnp.zeros((_NB, _D), jnp.float32)
    for i in range(_AF):
        xi = x_ref[:, i:i + 1]
        oh = (xi == lax.broadcasted_iota(jnp.int32, (1, _AV), 1)).astype(jnp.float32)
        h = h + jnp.dot(oh, aemb_ref[i], preferred_element_type=jnp.float32,
                        precision=lax.Precision.HIGHEST)
    h_ref[...] = h

    @pl.when(pl.program_id(0) == 0)
    def _tables():
        combo_ref[...] = ea_ref[0] * 64 + ea_ref[1] * 8 + ea_ref[2]
        cidx = lax.broadcasted_iota(jnp.int32, (512, 1), 0)
        for l in range(_L):
            acc = jnp.zeros((512, _D), jnp.float32)
            for j in range(_BF):
                shift = 3 * (_BF - 1 - j)
                digit = (cidx >> shift) & 7
                term = jnp.zeros((512, _D), jnp.float32)
                for v in range(_BV):
                    term = jnp.where(digit == v, bemb_ref[l, j, v][None, :], term)
                acc = acc + term
            ctab_ref[l] = acc


_encode_call = pl.pallas_call(
    _encode_body,
    grid=(_N // _NB,),
    in_specs=[
        pl.BlockSpec((_NB, _AF), lambda i: (i, 0)),
        pl.BlockSpec((_BF, _NCHUNKS, _C), lambda i: (0, 0, 0)),
        pl.BlockSpec((_AF, _AV, _D), lambda i: (0, 0, 0)),
        pl.BlockSpec((_L, _BF, _BV, _D), lambda i: (0, 0, 0, 0)),
    ],
    out_specs=(
        pl.BlockSpec((_NB, _D), lambda i: (i, 0)),
        pl.BlockSpec((_NCHUNKS, _C), lambda i: (0, 0)),
        pl.BlockSpec((_L, 512, _D), lambda i: (0, 0, 0)),
    ),
    out_shape=(
        jax.ShapeDtypeStruct((_N, _D), jnp.float32),
        jax.ShapeDtypeStruct((_NCHUNKS, _C), jnp.int32),
        jax.ShapeDtypeStruct((_L, 512, _D), jnp.float32),
    ),
)


def _relu_rows(rows):
    def row_body(i, carry):
        for g in range(8):
            sl = pl.ds(g * 16, 16)
            rows[i, sl] = jnp.maximum(rows[i, sl], 0.0)
        return carry
    lax.fori_loop(0, _C, row_body, 0)


_SB = 8
_NSB = _CPW // _SB


def _edge_body(h_hbm, src_hbm, dst_hbm, combo_hbm, ctab_hbm, zeros_hbm, out_hbm,
               srcidx_v, dstidx_v, comboidx_v, rows0, rows1,
               agg_sh, gsem0, gsem1, csem, ssem0, ssem1, isem):
    c = lax.axis_index("c")
    s = lax.axis_index("s")
    wid = s * 2 + c
    tid = s
    base = wid * _CPW

    rows = (rows0, rows1)
    gsems = (gsem0, gsem1)
    ssems = (ssem0, ssem1)

    pltpu.sync_copy(src_hbm.at[pl.ds(base, _SB)], srcidx_v.at[0])
    pltpu.sync_copy(dst_hbm.at[pl.ds(base, _SB)], dstidx_v.at[0])
    pltpu.sync_copy(combo_hbm.at[pl.ds(base, _SB)], comboidx_v.at[0])
    pltpu.sync_copy(zeros_hbm.at[pl.ds(tid * _RPT, _RPT)],
                    agg_sh.at[pl.ds(tid * _RPT, _RPT)])
    plsc.subcore_barrier()

    pltpu.async_copy(h_hbm.at[srcidx_v.at[0, 0]], rows[0], gsems[0])

    def sb_body(m, carry):
        mb = m % 2
        nb = 1 - mb

        @pl.when(m + 1 < _NSB)
        def _():
            off = base + (m + 1) * _SB
            pltpu.async_copy(src_hbm.at[pl.ds(off, _SB)], srcidx_v.at[nb], isem)
            pltpu.async_copy(dst_hbm.at[pl.ds(off, _SB)], dstidx_v.at[nb], isem)
            pltpu.async_copy(combo_hbm.at[pl.ds(off, _SB)], comboidx_v.at[nb], isem)

        for j in range(_SB):
            k = m * _SB + j
            cur = j % 2
            nxt = 1 - cur
            pltpu.make_async_copy(h_hbm.at[srcidx_v.at[mb, j]], rows[cur],
                                  gsems[cur]).wait()
            gadd = pltpu.async_copy(ctab_hbm.at[comboidx_v.at[mb, j]], rows[cur],
                                    csem, add=True)

            @pl.when(k >= 1)
            def _():
                pltpu.make_async_copy(rows[nxt], agg_sh.at[dstidx_v.at[0, 0]],
                                      ssems[nxt]).wait()

            if j + 1 < _SB:
                pltpu.async_copy(h_hbm.at[srcidx_v.at[mb, j + 1]], rows[nxt],
                                 gsems[nxt])
            else:
                @pl.when(k + 1 < _CPW)
                def _():
                    pltpu.make_async_copy(src_hbm.at[pl.ds(0, _SB)],
                                          srcidx_v.at[nb], isem).wait()
                    pltpu.make_async_copy(dst_hbm.at[pl.ds(0, _SB)],
                                          dstidx_v.at[nb], isem).wait()
                    pltpu.make_async_copy(combo_hbm.at[pl.ds(0, _SB)],
                                          comboidx_v.at[nb], isem).wait()
                    pltpu.async_copy(h_hbm.at[srcidx_v.at[nb, 0]], rows[nxt],
                                     gsems[nxt])
            gadd.wait()
            _relu_rows(rows[cur])
            pltpu.async_copy(rows[cur], agg_sh.at[dstidx_v.at[mb, j]],
                             ssems[cur])
        return carry

    lax.fori_loop(0, _NSB, sb_body, 0)
    pltpu.make_async_copy(rows[1], agg_sh.at[dstidx_v.at[0, 0]],
                          ssems[1]).wait()
    plsc.subcore_barrier()

    pltpu.sync_copy(agg_sh.at[pl.ds(tid * _RPT, _RPT)],
                    out_hbm.at[c, pl.ds(tid * _RPT, _RPT)])


@functools.cache
def _edge_call():
    return pl.kernel(
        _edge_body,
        out_type=jax.ShapeDtypeStruct((2, _NP, _D), jnp.float32),
        mesh=plsc.VectorSubcoreMesh(core_axis_name="c", subcore_axis_name="s"),
        scratch_types=[
            pltpu.VMEM((2, _SB, _C), jnp.int32),
            pltpu.VMEM((2, _SB, _C), jnp.int32),
            pltpu.VMEM((2, _SB, _C), jnp.int32),
            pltpu.VMEM((_C, _D), jnp.float32),
            pltpu.VMEM((_C, _D), jnp.float32),
            pltpu.VMEM_SHARED((_NP, _D), jnp.float32),
            pltpu.SemaphoreType.DMA,
            pltpu.SemaphoreType.DMA,
            pltpu.SemaphoreType.DMA,
            pltpu.SemaphoreType.DMA,
            pltpu.SemaphoreType.DMA,
            pltpu.SemaphoreType.DMA,
        ],    )


def _mlp_body(relu_out, scale_ref, h_ref, a0_ref, a1_ref, w1_ref, b1_ref,
              g1_ref, be1_ref, w2_ref, b2_ref, bng_ref, bnbe_ref, out_ref):
    z = scale_ref[...] * h_ref[...] + a0_ref[...] + a1_ref[...]
    z = jnp.dot(z, w1_ref[...], preferred_element_type=jnp.float32) + b1_ref[...]
    mu = jnp.mean(z, axis=0, keepdims=True)
    zc = z - mu
    var = jnp.mean(zc * zc, axis=0, keepdims=True)
    z = g1_ref[...] * zc * lax.rsqrt(var + 1e-5) + be1_ref[...]
    z = jnp.maximum(z, 0.0)
    z = jnp.dot(z, w2_ref[...], preferred_element_type=jnp.float32) + b2_ref[...]
    mu2 = jnp.mean(z, axis=0, keepdims=True)
    zc2 = z - mu2
    var2 = jnp.mean(zc2 * zc2, axis=0, keepdims=True)
    z = bng_ref[...] * zc2 * lax.rsqrt(var2 + 1e-5) + bnbe_ref[...]
    if relu_out:
        z = jnp.maximum(z, 0.0)
    out_ref[...] = z


def _mlp_call(relu_out):
    return pl.pallas_call(
        functools.partial(_mlp_body, relu_out),
        out_shape=jax.ShapeDtypeStruct((_N, _D), jnp.float32),
    )


def kernel(x, edge_index, edge_attr, batch, atom_emb, bond_emb, W1, b1, g1,
           be1, W2, b2, eps_p, bn_g, bn_be):
    del batch
    ea3 = edge_attr.T.reshape(_BF, _NCHUNKS, _C)
    h, combo2d, ctab = _encode_call(x, ea3, atom_emb, bond_emb)
    pad = _EP - _E
    pidx = jnp.arange(pad, dtype=jnp.int32)
    combo = jnp.concatenate([combo2d.reshape(_E), pidx % 512]).reshape(_NCH_P, _C)
    src = jnp.concatenate([edge_index[0], pidx % _N]).reshape(_NCH_P, _C)
    dst = jnp.concatenate([edge_index[1], _N + pidx % (_NP - _N)]).reshape(_NCH_P, _C)
    zeros = jnp.zeros((_NP, _D), jnp.float32)
    for l in range(_L):
        agg2 = _edge_call()(h, src, dst, combo, ctab[l], zeros)
        scale = (1.0 + eps_p[l]).reshape(1, 1)
        h = _mlp_call(l < _L - 1)(
            scale, h, agg2[0, :_N], agg2[1, :_N], W1[l], b1[l].reshape(1, _D),
            g1[l].reshape(1, _D), be1[l].reshape(1, _D), W2[l],
            b2[l].reshape(1, _D), bn_g[l].reshape(1, _D), bn_be[l].reshape(1, _D))
    return h
